# baseline (speedup 1.0000x reference)
import functools

import jax
import jax.numpy as jnp
from jax import lax
from jax.experimental import pallas as pl
from jax.experimental.pallas import tpu as pltpu
from jax.experimental.pallas import tpu_sc as plsc

N = 10000
E = 320000
D_HID = 128
D_OUT = 64

NC, NS = 2, 16
EPT = E // (NC * NS)
K = 80
NCHUNK = EPT // K
NP = 10240
ROWS_PT = NP // NS
DB = NP // 128

_EPS = 1e-5


def _make_agg(d, with_deg):
    mesh = plsc.VectorSubcoreMesh(core_axis_name="c", subcore_axis_name="s")
    out_type = jax.ShapeDtypeStruct((NC, NP, d), jnp.float32)
    NB = 2 if with_deg else 3
    NI = 4
    scratch = [
        pltpu.VMEM((EPT,), jnp.int32),
        pltpu.VMEM((NI, K), jnp.int32),
        pltpu.VMEM((NB, K, d), jnp.float32),
        pltpu.VMEM_SHARED((NP, d), jnp.float32),
        pltpu.SemaphoreType.DMA((NB,)),
        pltpu.SemaphoreType.DMA((NB,)),
        pltpu.SemaphoreType.DMA((NI,)),
    ]
    if with_deg:
        out_type = (out_type,
                    jax.ShapeDtypeStruct((NC, NS, NP), jnp.float32),
                    jax.ShapeDtypeStruct((NC, NP), jnp.float32))
        scratch += [
            pltpu.VMEM((NP,), jnp.float32),
            pltpu.VMEM((NP // NS,), jnp.float32),
            pltpu.VMEM((NP // NS,), jnp.float32),
        ]

    @functools.partial(
        pl.kernel, out_type=out_type, mesh=mesh, scratch_types=scratch,
        compiler_params=pltpu.CompilerParams(
            needs_layout_passes=False,
            use_tc_tiling_on_sc=(d % 128 == 0)))
    def agg(xt_hbm, edge_hbm, zero_hbm, *rest):
        if with_deg:
            (out_hbm, dscr_hbm, deg_hbm,
             srcv, dstv, rows, acc, sem_g, sem_s, sem_i,
             degv, dtmp, dred) = rest
        else:
            out_hbm, srcv, dstv, rows, acc, sem_g, sem_s, sem_i = rest
        c = lax.axis_index("c")
        s = lax.axis_index("s")
        wid = c * NS + s
        base = wid * EPT
        pltpu.sync_copy(edge_hbm.at[pl.ds(base, EPT)], srcv)
        pltpu.sync_copy(zero_hbm.at[pl.ds(s * ROWS_PT, ROWS_PT)],
                        acc.at[pl.ds(s * ROWS_PT, ROWS_PT)])
        if with_deg:
            @pl.loop(0, NP // 16)
            def zdeg(i):
                degv[pl.ds(i * 16, 16)] = jnp.zeros((16,), jnp.float32)
        plsc.subcore_barrier()

        def start_gather(j, b):
            pltpu.async_copy(xt_hbm.at[srcv.at[pl.ds(j * K, K)]],
                             rows.at[b], sem_g.at[b])

        def start_idx(j):
            bi = j % NI if isinstance(j, int) else lax.rem(j, NI)
            pltpu.async_copy(edge_hbm.at[pl.ds(E + base + j * K, K)],
                             dstv.at[bi], sem_i.at[bi])

        def wait_dma(sem_b):
            pltpu.make_async_copy(xt_hbm.at[pl.ds(0, K)], rows.at[0],
                                  sem_b).wait()

        def wait_idx(bi):
            pltpu.make_async_copy(edge_hbm.at[pl.ds(0, K)], dstv.at[0],
                                  sem_i.at[bi]).wait()

        def do_chunk(j, b, bi, wait_other, prefetch, pre_idx):
            wait_dma(sem_g.at[b])
            wait_idx(bi)
            pltpu.async_copy(rows.at[b], acc.at[dstv.at[bi]], sem_s.at[b],
                             add=True)
            if prefetch:
                if wait_other:
                    wait_dma(sem_s.at[1 - b])
                start_gather(j + 1, 1 - b)
                if pre_idx:
                    start_idx(j + 3)
            if with_deg:
                ones = jnp.ones((16,), jnp.float32)
                for i in range(K // 16):
                    d16 = dstv[bi, pl.ds(i * 16, 16)]
                    plsc.addupdate_scatter(degv, [d16], ones)

        if NB == 2:
            for j in range(3):
                start_idx(j)
            start_gather(0, 0)
            do_chunk(0, 0, 0, False, True, True)

            @pl.loop(0, (NCHUNK - 5) // 2)
            def grp(g):
                j = 1 + 2 * g
                do_chunk(j, 1, lax.rem(j, NI), True, True, True)
                do_chunk(j + 1, 0, lax.rem(j + 1, NI), True, True, True)

            do_chunk(NCHUNK - 4, 1, (NCHUNK - 4) % NI, True, True, True)
            do_chunk(NCHUNK - 3, 0, (NCHUNK - 3) % NI, True, True, False)
            do_chunk(NCHUNK - 2, 1, (NCHUNK - 2) % NI, True, True, False)
            do_chunk(NCHUNK - 1, 0, (NCHUNK - 1) % NI, True, False, False)
            wait_dma(sem_s.at[1])
            wait_dma(sem_s.at[0])
        else:
            def do3(j, wait_s, prefetch, pre_idx):
                b = lax.rem(j, NB) if not isinstance(j, int) else j % NB
                bi = lax.rem(j, NI) if not isinstance(j, int) else j % NI
                wait_dma(sem_g.at[b])
                wait_idx(bi)
                pltpu.async_copy(rows.at[b], acc.at[dstv.at[bi]],
                                 sem_s.at[b], add=True)
                if prefetch:
                    b1 = (lax.rem(j + 1, NB) if not isinstance(j, int)
                          else (j + 1) % NB)
                    if wait_s:
                        wait_dma(sem_s.at[b1])
                    start_gather(j + 1, b1)
                if pre_idx:
                    start_idx(j + 2)

            start_idx(0)
            start_idx(1)
            start_gather(0, 0)
            do3(0, False, True, True)
            do3(1, False, True, True)

            @pl.loop(2, NCHUNK - 3)
            def grp(j):
                do3(j, True, True, True)

            do3(NCHUNK - 3, True, True, True)
            do3(NCHUNK - 2, True, True, False)
            do3(NCHUNK - 1, True, False, False)
            wait_dma(sem_s.at[(NCHUNK - 3) % NB])
            wait_dma(sem_s.at[(NCHUNK - 2) % NB])
            wait_dma(sem_s.at[(NCHUNK - 1) % NB])

        if with_deg:
            pltpu.sync_copy(degv, dscr_hbm.at[c, s])
        plsc.subcore_barrier()
        pltpu.sync_copy(acc.at[pl.ds(s * ROWS_PT, ROWS_PT)],
                        out_hbm.at[c].at[pl.ds(s * ROWS_PT, ROWS_PT)])
        if with_deg:
            stripe = NP // NS
            pltpu.sync_copy(dscr_hbm.at[c, 0, pl.ds(s * stripe, stripe)], dred)
            for t in range(1, NS):
                pltpu.sync_copy(dscr_hbm.at[c, t, pl.ds(s * stripe, stripe)],
                                dtmp)

                @pl.loop(0, stripe // 16)
                def madd(j):
                    sl = pl.ds(j * 16, 16)
                    dred[sl] = dred[sl] + dtmp[sl]

            pltpu.sync_copy(dred, deg_hbm.at[c, pl.ds(s * stripe, stripe)])

    return agg


_agg1 = _make_agg(D_HID, True)
_agg2 = _make_agg(D_HID, False)
_agg3 = _make_agg(D_OUT, False)


def _ln(x, g, b):
    mean = jnp.mean(x, axis=-1, keepdims=True)
    var = jnp.mean((x - mean) ** 2, axis=-1, keepdims=True)
    return (x - mean) * lax.rsqrt(var + _EPS) * g + b


def _mm0_body(x_ref, w_ref, b_ref, o_ref):
    o_ref[...] = jnp.dot(x_ref[...], w_ref[...],
                         preferred_element_type=jnp.float32) + b_ref[...]


def _mid1_body(p_ref, dega_ref, degb_ref, g_ref, be_ref, w_ref, b_ref,
               o_ref, rdeg_ref):
    deg = dega_ref[...] + degb_ref[...]
    rdeg = 1.0 / jnp.where(deg == 0.0, 1.0, deg)
    x = (p_ref[0, :N] + p_ref[1, :N]) * rdeg
    x = jnp.maximum(_ln(x, g_ref[...], be_ref[...]), 0.0)
    o_ref[...] = jnp.dot(x, w_ref[...],
                         preferred_element_type=jnp.float32) + b_ref[...]
    rdeg_ref[...] = rdeg


def _mid2_body(p_ref, rdeg_ref, g_ref, be_ref, w_ref, b_ref, o_ref):
    x = (p_ref[0, :N] + p_ref[1, :N]) * rdeg_ref[...]
    x = jnp.maximum(_ln(x, g_ref[...], be_ref[...]), 0.0)
    o_ref[...] = jnp.dot(x, w_ref[...],
                         preferred_element_type=jnp.float32) + b_ref[...]


def _fin_body(p_ref, rdeg_ref, g_ref, be_ref, o_ref):
    x = (p_ref[0, :N] + p_ref[1, :N]) * rdeg_ref[...]
    o_ref[...] = _ln(x, g_ref[...], be_ref[...])


def _tc_call(body, out_shape):
    return pl.pallas_call(body, out_shape=out_shape)


def kernel(node_features, edge_index, W0, b0, W1, b1, W2, b2,
           g0, be0, g1, be1, g2, be2):
    edge = edge_index.astype(jnp.int32).reshape(2 * E)
    zh = jnp.zeros((NP, D_HID), jnp.float32)
    zo = jnp.zeros((NP, D_OUT), jnp.float32)

    f32 = jnp.float32
    xt0 = _tc_call(_mm0_body, jax.ShapeDtypeStruct((N, D_HID), f32))(
        node_features, W0, b0.reshape(1, -1))
    p0, _, deg = _agg1(xt0, edge, zh)
    dega = deg[0, :N].reshape(N, 1)
    degb = deg[1, :N].reshape(N, 1)

    xt1, rdeg = _tc_call(
        _mid1_body,
        (jax.ShapeDtypeStruct((N, D_HID), f32),
         jax.ShapeDtypeStruct((N, 1), f32)))(
        p0, dega, degb, g0.reshape(1, -1), be0.reshape(1, -1),
        W1, b1.reshape(1, -1))
    p1 = _agg2(xt1, edge, zh)

    xt2 = _tc_call(_mid2_body, jax.ShapeDtypeStruct((N, D_OUT), f32))(
        p1, rdeg, g1.reshape(1, -1), be1.reshape(1, -1), W2, b2.reshape(1, -1))
    p2 = _agg3(xt2, edge, zo)

    out = _tc_call(_fin_body, jax.ShapeDtypeStruct((N, D_OUT), f32))(
        p2, rdeg, g2.reshape(1, -1), be2.reshape(1, -1))
    return out
